# Initial kernel scaffold; baseline (speedup 1.0000x reference)
import jax
import jax.numpy as jnp
from jax.experimental import pallas as pl


def kernel(prot_x, prot_pos, lig_x, lig_pos, cross_rbf, params):
    raise NotImplementedError("write your pallas kernel here")


# speedup vs baseline: 4.6962x; 4.6962x over previous
import math
import functools

import jax
import jax.numpy as jnp
from jax import lax
from jax.experimental import pallas as pl

_ATOM_DIM = 18
_HID = 128
_N_LAYERS = 4
_CUTOFF = 8.0
_HEADS = 4
_RBF = 50
_MAXN = 32
_P = 512
_L = 48


def _silu(x):
    return x * jax.nn.sigmoid(x)


def _graph_body(n, pos_ref, posT_ref, col_ref, cnt_ref):
    iota = lax.broadcasted_iota(jnp.int32, (n, n), 1)
    d2 = jnp.zeros((n, n), jnp.float32)
    for c in range(3):
        diff = pos_ref[:, c:c + 1] - posT_ref[c:c + 1, :]
        d2 = d2 + diff * diff
    d = jnp.sqrt(d2)
    valid = (d < _CUTOFF) & (d > 0.0)
    count = jnp.sum(valid.astype(jnp.int32), axis=1, keepdims=True)
    keysA = jnp.where(valid, iota, n)
    keysB = jnp.where(valid, d, jnp.inf)
    self_idx = lax.broadcasted_iota(jnp.int32, (n, 1), 0)
    trunc = count > _MAXN
    kmax = jnp.minimum(count, _MAXN)
    cols = []
    for k in range(_MAXN):
        mA = jnp.min(keysA, axis=1, keepdims=True)
        keysA = jnp.where(keysA == mA, jnp.int32(1 << 30), keysA)
        mB = jnp.min(keysB, axis=1, keepdims=True)
        idxB = jnp.min(jnp.where(keysB == mB, iota, n), axis=1, keepdims=True)
        keysB = jnp.where(iota == idxB, jnp.inf, keysB)
        sel = jnp.where(trunc, idxB, mA)
        mk = k < kmax
        cols.append(jnp.where(mk, sel, self_idx))
    col_ref[...] = jnp.concatenate(cols, axis=1)
    cnt_ref[...] = count


def _graph(pos):
    n = pos.shape[0]
    return pl.pallas_call(
        functools.partial(_graph_body, n),
        out_shape=(
            jax.ShapeDtypeStruct((n, _MAXN), jnp.int32),
            jax.ShapeDtypeStruct((n, 1), jnp.int32),
        ),
    )(pos, pos.T)


def _encoder_body(n, x_ref, pos_ref, colT_ref, cnt_ref, wi_ref, bi_ref,
                  *rest):
    wrefs = rest[:-1]
    h_out_ref = rest[-1]

    h = jnp.dot(x_ref[...], wi_ref[...], preferred_element_type=jnp.float32)
    h = h + bi_ref[...]
    pos = pos_ref[...]
    kmin = jnp.minimum(cnt_ref[...], _MAXN)
    iota_col = lax.broadcasted_iota(jnp.int32, (n, 1), 0)

    for l in range(_N_LAYERS):
        (e0w, e0b, e1w, e1b, n0w, n0b, n1w, n1b, c0w, c0b, c1w) = (
            r[...] for r in wrefs[l * 11:(l + 1) * 11])
        A = jnp.dot(h, e0w[:_HID], preferred_element_type=jnp.float32) + e0b
        B = jnp.dot(h, e0w[_HID:2 * _HID], preferred_element_type=jnp.float32)
        wd = e0w[2 * _HID:2 * _HID + 1, :]

        def slot_step(k, carry):
            agg, pd = carry
            ck = colT_ref[pl.ds(k, 1), :]
            ohT = (iota_col == ck).astype(jnp.float32)
            Bk = lax.dot_general(ohT, B, (((0,), (0,)), ((), ())))
            posk = lax.dot_general(ohT, pos, (((0,), (0,)), ((), ())))
            diff = pos - posk
            dist = jnp.sqrt(jnp.sum(diff * diff, axis=1, keepdims=True))
            dist = jnp.maximum(dist, 1e-6)
            pre = A + Bk + dist * wd
            msg = _silu(jnp.dot(_silu(pre), e1w,
                                preferred_element_type=jnp.float32) + e1b)
            mk = (k < kmin).astype(jnp.float32)
            msg = msg * mk
            c0o = _silu(jnp.dot(msg, c0w,
                                preferred_element_type=jnp.float32) + c0b)
            cw = jnp.dot(c0o, c1w, preferred_element_type=jnp.float32)
            cw = jnp.clip(cw, -1.0, 1.0)
            return (agg + msg, pd + diff / dist * cw * mk)

        agg, pd = lax.fori_loop(
            0, _MAXN, slot_step,
            (jnp.zeros((n, _HID), jnp.float32), jnp.zeros((n, 3), jnp.float32)))
        nh = _silu(jnp.dot(h, n0w[:_HID], preferred_element_type=jnp.float32)
                   + jnp.dot(agg, n0w[_HID:],
                             preferred_element_type=jnp.float32) + n0b)
        h = h + (jnp.dot(nh, n1w, preferred_element_type=jnp.float32) + n1b)
        pos = pos + pd
    h_out_ref[...] = h


def _enc_weights(p):
    out = [p['inp']['w'], p['inp']['b'].reshape(1, _HID)]
    for lp in p['layers']:
        out += [lp['e0']['w'], lp['e0']['b'].reshape(1, _HID),
                lp['e1']['w'], lp['e1']['b'].reshape(1, _HID),
                lp['n0']['w'], lp['n0']['b'].reshape(1, _HID),
                lp['n1']['w'], lp['n1']['b'].reshape(1, _HID),
                lp['c0']['w'], lp['c0']['b'].reshape(1, _HID),
                lp['c1']['w']]
    return out


def _encode(x, pos, col, cnt, p):
    n = x.shape[0]
    args = [x, pos, col.T, cnt]
    args += _enc_weights(p)
    return pl.pallas_call(
        functools.partial(_encoder_body, n),
        out_shape=jax.ShapeDtypeStruct((n, _HID), jnp.float32),
    )(*args)


def _heads_body(lcb, ligh_ref, proth_ref, rbf_ref,
                wq, bq, wk, bk, wv, bv, wr, br, wo, bo, lng, lnb,
                a0w, a0b, a1w, a1b, p0w, p0b, p1w, p1b, i0w, i0b, i1w, i1b,
                pkd_ref, pose_ref, inter_ref, attnw_ref):
    L, P, H, D = _L, _P, _HEADS, _HID
    d = D // H
    nchunks = L // lcb
    ligh = ligh_ref[...]
    proth = proth_ref[...]
    Q = jnp.dot(ligh, wq[...], preferred_element_type=jnp.float32) + bq[...]
    K = jnp.dot(proth, wk[...], preferred_element_type=jnp.float32) + bk[...]
    V = jnp.dot(proth, wv[...], preferred_element_type=jnp.float32) + bv[...]
    S = (lax.broadcasted_iota(jnp.int32, (D, H), 0) // d
         == lax.broadcasted_iota(jnp.int32, (D, H), 1)).astype(jnp.float32)
    Kexp = jnp.broadcast_to(K[None, :, :], (lcb, P, D)).reshape(lcb * P, D)
    Vexp = jnp.broadcast_to(V[None, :, :], (lcb, P, D)).reshape(lcb * P, D)
    inv_sqrt_d = 1.0 / math.sqrt(d)
    PB = jnp.dot(proth, i0w[D:2 * D], preferred_element_type=jnp.float32)
    PBexp = jnp.broadcast_to(PB[None, :, :], (lcb, P, D)).reshape(lcb * P, D)

    att_rows = []
    attnw_rows = []
    probs = []
    for ci in range(nchunks):
        ls = ci * lcb
        rs = ls * P
        E = lcb * P
        Qexp = jnp.broadcast_to(Q[ls:ls + lcb][:, None, :],
                                (lcb, P, D)).reshape(E, D)
        rbfc = rbf_ref[rs:rs + E, :]
        sc = jnp.dot(Qexp * Kexp, S,
                     preferred_element_type=jnp.float32) * inv_sqrt_d
        sc = sc + jnp.dot(rbfc, wr[...],
                          preferred_element_type=jnp.float32) + br[...]
        sc3 = sc.reshape(lcb, P, H)
        mx = jnp.max(sc3, axis=1, keepdims=True)
        ex = jnp.exp(sc3 - mx)
        pr3 = ex / jnp.sum(ex, axis=1, keepdims=True)
        attnw_rows.append(jnp.mean(pr3, axis=2))
        probs.append(pr3)
        Pexp = jnp.dot(pr3.reshape(E, H), S.T,
                       preferred_element_type=jnp.float32)
        att_rows.append(jnp.sum((Pexp * Vexp).reshape(lcb, P, D), axis=1))
    att = jnp.concatenate(att_rows, axis=0) if nchunks > 1 else att_rows[0]
    attnw_ref[...] = (jnp.concatenate(attnw_rows, axis=0)
                      if nchunks > 1 else attnw_rows[0])
    out = jnp.dot(att, wo[...], preferred_element_type=jnp.float32) + bo[...]
    xres = ligh + out
    mu = jnp.mean(xres, axis=1, keepdims=True)
    var = jnp.mean((xres - mu) * (xres - mu), axis=1, keepdims=True)
    lig_att = (xres - mu) / jnp.sqrt(var + 1e-5) * lng[...] + lnb[...]

    cr = jnp.mean(lig_att, axis=0, keepdims=True)
    pkd_ref[...] = jnp.dot(
        _silu(jnp.dot(cr, a0w[...], preferred_element_type=jnp.float32)
              + a0b[...]), a1w[...],
        preferred_element_type=jnp.float32) + a1b[...]
    pose_ref[...] = jax.nn.sigmoid(jnp.dot(
        _silu(jnp.dot(cr, p0w[...], preferred_element_type=jnp.float32)
              + p0b[...]), p1w[...],
        preferred_element_type=jnp.float32) + p1b[...])

    LA = jnp.dot(lig_att, i0w[:D], preferred_element_type=jnp.float32)
    for ci in range(nchunks):
        ls = ci * lcb
        rs = ls * P
        E = lcb * P
        rbfc = rbf_ref[rs:rs + E, :]
        LAexp = jnp.broadcast_to(LA[ls:ls + lcb][:, None, :],
                                 (lcb, P, D)).reshape(E, D)
        pre = LAexp + PBexp + jnp.dot(
            rbfc, i0w[2 * D:], preferred_element_type=jnp.float32) + i0b[...]
        inter_ref[rs:rs + E, :] = jax.nn.sigmoid(
            jnp.dot(_silu(pre), i1w[...],
                    preferred_element_type=jnp.float32) + i1b[...])


def _heads(lig_h, prot_h, rbf2, p, lcb):
    args = [lig_h, prot_h, rbf2,
            p['q']['w'], p['q']['b'].reshape(1, _HID),
            p['k']['w'], p['k']['b'].reshape(1, _HID),
            p['v']['w'], p['v']['b'].reshape(1, _HID),
            p['rbf']['w'], p['rbf']['b'].reshape(1, _HEADS),
            p['o']['w'], p['o']['b'].reshape(1, _HID),
            p['ln_g'].reshape(1, _HID), p['ln_b'].reshape(1, _HID),
            p['aff0']['w'], p['aff0']['b'].reshape(1, _HID),
            p['aff1']['w'], p['aff1']['b'].reshape(1, 1),
            p['pose0']['w'], p['pose0']['b'].reshape(1, _HID // 2),
            p['pose1']['w'], p['pose1']['b'].reshape(1, 1),
            p['int0']['w'], p['int0']['b'].reshape(1, _HID),
            p['int1']['w'], p['int1']['b'].reshape(1, 5)]
    return pl.pallas_call(
        functools.partial(_heads_body, lcb),
        out_shape=(
            jax.ShapeDtypeStruct((1, 1), jnp.float32),
            jax.ShapeDtypeStruct((1, 1), jnp.float32),
            jax.ShapeDtypeStruct((_L * _P, 5), jnp.float32),
            jax.ShapeDtypeStruct((_L, _P), jnp.float32),
        ),
    )(*args)


def kernel(prot_x, prot_pos, lig_x, lig_pos, cross_rbf, params):
    prot_col, prot_cnt = _graph(prot_pos)
    lig_col, lig_cnt = _graph(lig_pos)
    prot_h = _encode(prot_x, prot_pos, prot_col, prot_cnt, params['prot'])
    lig_h = _encode(lig_x, lig_pos, lig_col, lig_cnt, params['lig'])
    rbf2 = cross_rbf.reshape(_L * _P, _RBF)
    pkd2, pose2, inter2, attnw = _heads(lig_h, prot_h, rbf2, params, lcb=12)
    return (pkd2.reshape(1), pose2.reshape(1),
            inter2.reshape(_L, _P, 5), attnw)


# speedup vs baseline: 5.1244x; 1.0912x over previous
import math
import functools

import jax
import jax.numpy as jnp
from jax import lax
from jax.experimental import pallas as pl

_ATOM_DIM = 18
_HID = 128
_N_LAYERS = 4
_CUTOFF = 8.0
_HEADS = 4
_RBF = 50
_MAXN = 32
_P = 512
_L = 48


def _silu(x):
    return x * jax.nn.sigmoid(x)


def _graph_body(n, pos_ref, posT_ref, col_ref, cnt_ref):
    iota = lax.broadcasted_iota(jnp.int32, (n, n), 1)
    d2 = jnp.zeros((n, n), jnp.float32)
    for c in range(3):
        diff = pos_ref[:, c:c + 1] - posT_ref[c:c + 1, :]
        d2 = d2 + diff * diff
    d = jnp.sqrt(d2)
    valid = (d < _CUTOFF) & (d > 0.0)
    count = jnp.sum(valid.astype(jnp.int32), axis=1, keepdims=True)
    keysA = jnp.where(valid, iota, n)
    keysB = jnp.where(valid, d, jnp.inf)
    self_idx = lax.broadcasted_iota(jnp.int32, (n, 1), 0)
    trunc = count > _MAXN
    kmax = jnp.minimum(count, _MAXN)
    cols = []
    for k in range(_MAXN):
        mA = jnp.min(keysA, axis=1, keepdims=True)
        keysA = jnp.where(keysA == mA, jnp.int32(1 << 30), keysA)
        mB = jnp.min(keysB, axis=1, keepdims=True)
        idxB = jnp.min(jnp.where(keysB == mB, iota, n), axis=1, keepdims=True)
        keysB = jnp.where(iota == idxB, jnp.inf, keysB)
        sel = jnp.where(trunc, idxB, mA)
        mk = k < kmax
        cols.append(jnp.where(mk, sel, self_idx))
    col_ref[...] = jnp.concatenate(cols, axis=1)
    cnt_ref[...] = count


def _graph(pos):
    n = pos.shape[0]
    return pl.pallas_call(
        functools.partial(_graph_body, n),
        out_shape=(
            jax.ShapeDtypeStruct((n, _MAXN), jnp.int32),
            jax.ShapeDtypeStruct((n, 1), jnp.int32),
        ),
    )(pos, pos.T)


def _hilo(x):
    hi = x.astype(jnp.bfloat16)
    lo = (x - hi.astype(jnp.float32)).astype(jnp.bfloat16)
    return hi, lo


_G = 4


def _encoder_body(n, x_ref, pos_ref, colg_ref, cnt_ref, wi_ref, bi_ref,
                  *rest):
    wrefs = rest[:-1]
    h_out_ref = rest[-1]

    h = jnp.dot(x_ref[...], wi_ref[...], preferred_element_type=jnp.float32)
    h = h + bi_ref[...]
    pos = pos_ref[...]
    kmin = jnp.minimum(cnt_ref[...], _MAXN)
    kmin_g = jnp.broadcast_to(kmin[None], (_G, n, 1)).reshape(_G * n, 1)
    kidx0 = lax.broadcasted_iota(jnp.int32, (_G, n, 1), 0).reshape(_G * n, 1)
    iota_col = lax.broadcasted_iota(jnp.int32, (n, 1), 0)

    for l in range(_N_LAYERS):
        (e0w, e0b, e1w, e1b, n0w, n0b, n1w, n1b, c0w, c0b, c1wr) = (
            r[...] for r in wrefs[l * 11:(l + 1) * 11])
        A = jnp.dot(h, e0w[:_HID], preferred_element_type=jnp.float32) + e0b
        B = jnp.dot(h, e0w[_HID:2 * _HID], preferred_element_type=jnp.float32)
        wd = e0w[2 * _HID:2 * _HID + 1, :]
        Bhi, Blo = _hilo(B)
        Phi, Plo = _hilo(pos)
        T = jnp.concatenate([Bhi, Blo, Phi, Plo], axis=1)
        wd_b = wd.astype(jnp.bfloat16).astype(jnp.float32)
        c1wr_b = c1wr.astype(jnp.bfloat16).astype(jnp.float32)
        Aexp = jnp.broadcast_to(A[None], (_G, n, _HID)).reshape(_G * n, _HID)
        posexp = jnp.broadcast_to(pos[None], (_G, n, 3)).reshape(_G * n, 3)

        def slot_step(r, carry):
            agg, pd = carry
            ck = colg_ref[pl.ds(r, 1), :]
            ohT = (iota_col == ck).astype(jnp.bfloat16)
            R = lax.dot_general(ohT, T, (((0,), (0,)), ((), ())),
                                preferred_element_type=jnp.float32)
            Bk = R[:, :_HID] + R[:, _HID:2 * _HID]
            posk = R[:, 2 * _HID:2 * _HID + 3] + R[:, 2 * _HID + 3:]
            diff = posexp - posk
            dist = jnp.sqrt(jnp.sum(diff * diff, axis=1, keepdims=True))
            dist = jnp.maximum(dist, 1e-6)
            dist_b = dist.astype(jnp.bfloat16).astype(jnp.float32)
            pre = Aexp + Bk + dist_b * wd_b
            msg = _silu(jnp.dot(_silu(pre), e1w,
                                preferred_element_type=jnp.float32) + e1b)
            mk = (kidx0 + r * _G < kmin_g).astype(jnp.float32)
            msg = msg * mk
            c0o = _silu(jnp.dot(msg, c0w,
                                preferred_element_type=jnp.float32) + c0b)
            c0o_b = c0o.astype(jnp.bfloat16).astype(jnp.float32)
            cw = jnp.sum(c0o_b * c1wr_b, axis=1, keepdims=True)
            cw = jnp.clip(cw, -1.0, 1.0)
            cd = diff / dist * cw * mk
            agg = agg + jnp.sum(msg.reshape(_G, n, _HID), axis=0)
            pd = pd + jnp.sum(cd.reshape(_G, n, 3), axis=0)
            return (agg, pd)

        agg, pd = lax.fori_loop(
            0, _MAXN // _G, slot_step,
            (jnp.zeros((n, _HID), jnp.float32), jnp.zeros((n, 3), jnp.float32)))
        nh = _silu(jnp.dot(h, n0w[:_HID], preferred_element_type=jnp.float32)
                   + jnp.dot(agg, n0w[_HID:],
                             preferred_element_type=jnp.float32) + n0b)
        h = h + (jnp.dot(nh, n1w, preferred_element_type=jnp.float32) + n1b)
        pos = pos + pd
    h_out_ref[...] = h


def _enc_weights(p):
    out = [p['inp']['w'], p['inp']['b'].reshape(1, _HID)]
    for lp in p['layers']:
        out += [lp['e0']['w'], lp['e0']['b'].reshape(1, _HID),
                lp['e1']['w'], lp['e1']['b'].reshape(1, _HID),
                lp['n0']['w'], lp['n0']['b'].reshape(1, _HID),
                lp['n1']['w'], lp['n1']['b'].reshape(1, _HID),
                lp['c0']['w'], lp['c0']['b'].reshape(1, _HID),
                lp['c1']['w'].reshape(1, _HID)]
    return out


def _encode(x, pos, col, cnt, p):
    n = x.shape[0]
    args = [x, pos, col.T.reshape(_MAXN // _G, _G * n), cnt]
    args += _enc_weights(p)
    return pl.pallas_call(
        functools.partial(_encoder_body, n),
        out_shape=jax.ShapeDtypeStruct((n, _HID), jnp.float32),
    )(*args)


def _heads_body(lcb, ligh_ref, proth_ref, rbf_ref,
                wq, bq, wk, bk, wv, bv, wr, br, wo, bo, lng, lnb,
                a0w, a0b, a1w, a1b, p0w, p0b, p1w, p1b, i0w, i0b, i1w, i1b,
                pkd_ref, pose_ref, inter_ref, attnw_ref):
    L, P, H, D = _L, _P, _HEADS, _HID
    d = D // H
    nchunks = L // lcb
    ligh = ligh_ref[...]
    proth = proth_ref[...]
    Q = jnp.dot(ligh, wq[...], preferred_element_type=jnp.float32) + bq[...]
    K = jnp.dot(proth, wk[...], preferred_element_type=jnp.float32) + bk[...]
    V = jnp.dot(proth, wv[...], preferred_element_type=jnp.float32) + bv[...]
    S = (lax.broadcasted_iota(jnp.int32, (D, H), 0) // d
         == lax.broadcasted_iota(jnp.int32, (D, H), 1)).astype(jnp.float32)
    Kexp = jnp.broadcast_to(K[None, :, :], (lcb, P, D)).reshape(lcb * P, D)
    Vexp = jnp.broadcast_to(V[None, :, :], (lcb, P, D)).reshape(lcb * P, D)
    inv_sqrt_d = 1.0 / math.sqrt(d)
    PB = jnp.dot(proth, i0w[D:2 * D], preferred_element_type=jnp.float32)
    PBexp = jnp.broadcast_to(PB[None, :, :], (lcb, P, D)).reshape(lcb * P, D)

    att_rows = []
    attnw_rows = []
    probs = []
    for ci in range(nchunks):
        ls = ci * lcb
        rs = ls * P
        E = lcb * P
        Qexp = jnp.broadcast_to(Q[ls:ls + lcb][:, None, :],
                                (lcb, P, D)).reshape(E, D)
        rbfc = rbf_ref[rs:rs + E, :]
        qk = (Qexp.astype(jnp.bfloat16).astype(jnp.float32)
              * Kexp.astype(jnp.bfloat16).astype(jnp.float32))
        sc = jnp.concatenate(
            [jnp.sum(qk[:, hh * d:(hh + 1) * d], axis=1, keepdims=True)
             for hh in range(H)], axis=1) / math.sqrt(d)
        sc = sc + jnp.dot(rbfc, wr[...],
                          preferred_element_type=jnp.float32) + br[...]
        sc3 = sc.reshape(lcb, P, H)
        mx = jnp.max(sc3, axis=1, keepdims=True)
        ex = jnp.exp(sc3 - mx)
        pr3 = ex / jnp.sum(ex, axis=1, keepdims=True)
        attnw_rows.append(jnp.mean(pr3, axis=2))
        probs.append(pr3)
        Pexp = jnp.dot(pr3.reshape(E, H), S.T,
                       preferred_element_type=jnp.float32,
                       precision=lax.Precision.HIGHEST)
        pv = (Pexp.astype(jnp.bfloat16).astype(jnp.float32)
              * Vexp.astype(jnp.bfloat16).astype(jnp.float32))
        att_rows.append(jnp.sum(pv.reshape(lcb, P, D), axis=1))
    att = jnp.concatenate(att_rows, axis=0) if nchunks > 1 else att_rows[0]
    attnw_ref[...] = (jnp.concatenate(attnw_rows, axis=0)
                      if nchunks > 1 else attnw_rows[0])
    out = jnp.dot(att, wo[...], preferred_element_type=jnp.float32) + bo[...]
    xres = ligh + out
    mu = jnp.mean(xres, axis=1, keepdims=True)
    var = jnp.mean((xres - mu) * (xres - mu), axis=1, keepdims=True)
    lig_att = (xres - mu) / jnp.sqrt(var + 1e-5) * lng[...] + lnb[...]

    cr = jnp.mean(lig_att, axis=0, keepdims=True)
    pkd_ref[...] = jnp.dot(
        _silu(jnp.dot(cr, a0w[...], preferred_element_type=jnp.float32)
              + a0b[...]), a1w[...],
        preferred_element_type=jnp.float32) + a1b[...]
    pose_ref[...] = jax.nn.sigmoid(jnp.dot(
        _silu(jnp.dot(cr, p0w[...], preferred_element_type=jnp.float32)
              + p0b[...]), p1w[...],
        preferred_element_type=jnp.float32) + p1b[...])

    LA = jnp.dot(lig_att, i0w[:D], preferred_element_type=jnp.float32)
    for ci in range(nchunks):
        ls = ci * lcb
        rs = ls * P
        E = lcb * P
        rbfc = rbf_ref[rs:rs + E, :]
        LAexp = jnp.broadcast_to(LA[ls:ls + lcb][:, None, :],
                                 (lcb, P, D)).reshape(E, D)
        pre = LAexp + PBexp + jnp.dot(
            rbfc, i0w[2 * D:], preferred_element_type=jnp.float32) + i0b[...]
        st = _silu(pre).astype(jnp.bfloat16).astype(jnp.float32)
        i1wT = i1w[...].astype(jnp.bfloat16).astype(jnp.float32)
        inter_ref[rs:rs + E, :] = jax.nn.sigmoid(jnp.concatenate(
            [jnp.sum(st * i1wT[c:c + 1, :], axis=1, keepdims=True)
             for c in range(5)], axis=1) + i1b[...])


def _heads(lig_h, prot_h, rbf2, p, lcb):
    args = [lig_h, prot_h, rbf2,
            p['q']['w'], p['q']['b'].reshape(1, _HID),
            p['k']['w'], p['k']['b'].reshape(1, _HID),
            p['v']['w'], p['v']['b'].reshape(1, _HID),
            p['rbf']['w'], p['rbf']['b'].reshape(1, _HEADS),
            p['o']['w'], p['o']['b'].reshape(1, _HID),
            p['ln_g'].reshape(1, _HID), p['ln_b'].reshape(1, _HID),
            p['aff0']['w'], p['aff0']['b'].reshape(1, _HID),
            p['aff1']['w'], p['aff1']['b'].reshape(1, 1),
            p['pose0']['w'], p['pose0']['b'].reshape(1, _HID // 2),
            p['pose1']['w'], p['pose1']['b'].reshape(1, 1),
            p['int0']['w'], p['int0']['b'].reshape(1, _HID),
            p['int1']['w'].T, p['int1']['b'].reshape(1, 5)]
    return pl.pallas_call(
        functools.partial(_heads_body, lcb),
        out_shape=(
            jax.ShapeDtypeStruct((1, 1), jnp.float32),
            jax.ShapeDtypeStruct((1, 1), jnp.float32),
            jax.ShapeDtypeStruct((_L * _P, 5), jnp.float32),
            jax.ShapeDtypeStruct((_L, _P), jnp.float32),
        ),
    )(*args)


def kernel(prot_x, prot_pos, lig_x, lig_pos, cross_rbf, params):
    prot_col, prot_cnt = _graph(prot_pos)
    lig_col, lig_cnt = _graph(lig_pos)
    prot_h = _encode(prot_x, prot_pos, prot_col, prot_cnt, params['prot'])
    lig_h = _encode(lig_x, lig_pos, lig_col, lig_cnt, params['lig'])
    rbf2 = cross_rbf.reshape(_L * _P, _RBF)
    pkd2, pose2, inter2, attnw = _heads(lig_h, prot_h, rbf2, params, lcb=12)
    return (pkd2.reshape(1), pose2.reshape(1),
            inter2.reshape(_L, _P, 5), attnw)
